# baseline (speedup 1.0000x reference)
import functools

import jax
import jax.numpy as jnp
from jax import lax
from jax.experimental import pallas as pl
from jax.experimental.pallas import tpu as pltpu
from jax.experimental.pallas import tpu_sc as plsc

B = 16384
D = 32
NC = 2
NS = 16
NW = NC * NS
BPW = B // NW
PASS = 256
NPASS = BPW // PASS
CH = 128
NCHUNK = PASS // CH
GROUPS = PASS // 16

_mesh = plsc.VectorSubcoreMesh(core_axis_name="c", subcore_axis_name="s")


@functools.partial(
    pl.kernel,
    mesh=_mesh,
    compiler_params=pltpu.CompilerParams(
        needs_layout_passes=False, use_tc_tiling_on_sc=True,
        disable_bounds_checks=True),
    out_type=jax.ShapeDtypeStruct((B,), jnp.float32),
    scratch_types=[
        pltpu.VMEM((BPW,), jnp.int32),
        pltpu.VMEM((BPW,), jnp.int32),
        pltpu.VMEM((PASS, 128), jnp.float32),
        pltpu.VMEM((PASS, 128), jnp.float32),
        pltpu.VMEM((48,), jnp.float32),
        pltpu.VMEM((BPW,), jnp.float32),
        pltpu.SemaphoreType.DMA,
    ],
)
def _gmf_sc(user_hbm, item_hbm, uemb_hbm, iemb_hbm, wb_hbm, out_hbm,
            uidx_v, iidx_v, urows_v, irows_v, wb_v, out_v, sem):
    wid = lax.axis_index("s") * NC + lax.axis_index("c")
    base = pl.multiple_of(wid * BPW, BPW)

    pltpu.sync_copy(user_hbm.at[pl.ds(base, BPW)], uidx_v)
    pltpu.sync_copy(item_hbm.at[pl.ds(base, BPW)], iidx_v)
    pltpu.sync_copy(wb_hbm, wb_v)

    w0 = wb_v[pl.ds(0, 16)]
    w1 = wb_v[pl.ds(16, 16)]
    bias = wb_v[pl.ds(32, 16)][0]
    lane = lax.iota(jnp.int32, 16)

    for p in range(NPASS):
        copies = []
        for j in range(NCHUNK):
            isl = pl.ds(p * PASS + j * CH, CH)
            bsl = pl.ds(j * CH, CH)
            copies.append(pltpu.make_async_copy(
                uemb_hbm.at[uidx_v.at[isl], pl.ds(0, 128)],
                urows_v.at[bsl], sem))
            copies.append(pltpu.make_async_copy(
                iemb_hbm.at[iidx_v.at[isl], pl.ds(0, 128)],
                irows_v.at[bsl], sem))
        for c in copies:
            c.start()
        for c in copies:
            c.wait()

        def compute_group(g, carry):
            row0 = pl.multiple_of(g * 16, 16)
            svec = jnp.zeros((16,), jnp.float32)
            for k in range(16):
                r = row0 + k
                u0 = urows_v[r, pl.ds(0, 16)]
                u1 = urows_v[r, pl.ds(16, 16)]
                i0 = irows_v[r, pl.ds(0, 16)]
                i1 = irows_v[r, pl.ds(16, 16)]
                s = jnp.sum(u0 * i0 * w0 + u1 * i1 * w1)
                svec = jnp.where(lane == k, s, svec)
            logits = svec + bias
            out_v[pl.ds(carry + row0, 16)] = 1.0 / (1.0 + jnp.exp(-logits))
            return carry

        lax.fori_loop(0, GROUPS, compute_group, p * PASS)

    pltpu.sync_copy(out_v, out_hbm.at[pl.ds(base, BPW)])


def kernel(user, item, user_emb, item_emb, out_w, out_b):
    wb = jnp.concatenate(
        [out_w.reshape(-1), out_b.reshape(-1),
         jnp.zeros((48 - D - 1,), jnp.float32)]).astype(jnp.float32)
    return _gmf_sc(user.astype(jnp.int32), item.astype(jnp.int32),
                   user_emb, item_emb, wb)
